# baseline (speedup 1.0000x reference)
import functools

import jax
import jax.numpy as jnp
from jax import lax
from jax.experimental import pallas as pl
from jax.experimental.pallas import tpu as pltpu
from jax.experimental.pallas import tpu_sc as plsc

_B, _S, _DIM = 4, 2048, 6
_VOCAB, _D = 512, 1024
_DP = _D // 2
_ROWS = _B * (_S - 1)
_BLK = 512
_NBLK = 16
_NPOS = _NBLK * _BLK
_NIDX = _NPOS * _DIM

_NC, _NS = 2, 16
_NW = _NC * _NS
_BPW = _NIDX // _NW
_CH = 128


def _sc_gather(table_hbm, idx_hbm, out_hbm, idx_v, rows_v, sem):
    wid = lax.axis_index("s") * _NC + lax.axis_index("c")
    base = wid * _BPW
    pltpu.sync_copy(idx_hbm.at[pl.ds(base, _BPW)], idx_v)
    for c in range(_BPW // _CH):
        pltpu.async_copy(
            table_hbm.at[idx_v.at[pl.ds(c * _CH, _CH)]], rows_v, sem
        ).wait()
        pltpu.sync_copy(rows_v, out_hbm.at[pl.ds(base + c * _CH, _CH)])


def _loss_kernel(g_ref, w_ref, out_ref):
    i = pl.program_id(0)

    he = jnp.zeros((_BLK, _DP), dtype=jnp.float32)
    ho = jnp.zeros((_BLK, _DP), dtype=jnp.float32)
    for d in range(_DIM):
        gd = g_ref[d * _BLK:(d + 1) * _BLK, :]
        he = he + lax.bitcast_convert_type(gd << 16, jnp.float32)
        ho = ho + lax.bitcast_convert_type(gd & jnp.int32(-65536),
                                           jnp.float32)

    t = jnp.concatenate([jnp.tanh(he), jnp.tanh(ho)], axis=1)
    o = jnp.dot(t.astype(jnp.bfloat16), w_ref[...],
                preferred_element_type=jnp.float32)
    diff = o - 1.0

    row = i * _BLK + jax.lax.broadcasted_iota(jnp.int32, (_BLK, _D), 0)
    diff = jnp.where(row < _ROWS, diff, 0.0)
    s = jnp.sum(diff * diff, keepdims=True)

    @pl.when(i == 0)
    def _():
        out_ref[...] = jnp.zeros((1, 1), jnp.float32)

    out_ref[...] += s


def kernel(x, emb, W):
    xi = x[:, :-1].reshape(_ROWS, _DIM).astype(jnp.int32)
    idx = jnp.pad(xi, ((0, _NPOS - _ROWS), (0, 0)))
    idx = idx + jnp.arange(_DIM, dtype=jnp.int32) * _VOCAB
    idx = idx.reshape(_NBLK, _BLK, _DIM).transpose(0, 2, 1).reshape(_NIDX)

    bits = lax.bitcast_convert_type(emb.reshape(_DIM * _VOCAB, _D),
                                    jnp.int32)
    table = (lax.shift_right_logical(bits[:, 0::2], 16)
             | (bits[:, 1::2] & jnp.int32(-65536)))

    w_bf = jnp.concatenate([W[0::2, :], W[1::2, :]], axis=0).astype(
        jnp.bfloat16)

    mesh = plsc.VectorSubcoreMesh(core_axis_name="c", subcore_axis_name="s")
    gather = functools.partial(
        pl.kernel,
        mesh=mesh,
        out_type=jax.ShapeDtypeStruct((_NIDX, _DP), jnp.int32),
        scratch_types=[
            pltpu.VMEM((_BPW,), jnp.int32),
            pltpu.VMEM((_CH, _DP), jnp.int32),
            pltpu.SemaphoreType.DMA,
        ],
    )(_sc_gather)
    g = gather(table, idx)

    out = pl.pallas_call(
        _loss_kernel,
        grid=(_NBLK,),
        in_specs=[
            pl.BlockSpec((_DIM * _BLK, _DP), lambda i: (i, 0)),
            pl.BlockSpec((_D, _D), lambda i: (0, 0)),
        ],
        out_specs=pl.BlockSpec((1, 1), lambda i: (0, 0)),
        out_shape=jax.ShapeDtypeStruct((1, 1), jnp.float32),
    )(g, w_bf)

    return out[0, 0] / (_ROWS * _D)


# speedup vs baseline: 3.6375x; 3.6375x over previous
import functools

import jax
import jax.numpy as jnp
from jax import lax
from jax.experimental import pallas as pl
from jax.experimental.pallas import tpu as pltpu
from jax.experimental.pallas import tpu_sc as plsc

_B, _S, _DIM = 4, 2048, 6
_VOCAB, _D = 512, 1024
_DP = _D // 2
_ROWS = _B * (_S - 1)
_BLK = 512
_NBLK = 16
_NPOS = _NBLK * _BLK
_NIDX = _NPOS * _DIM

_NC, _NS = 2, 16
_NW = _NC * _NS
_BPW = _NIDX // _NW
_CH = 96
_NCH = _BPW // _CH


def _sc_gather(table_hbm, idx_hbm, out_hbm, idx_v, rows_a, rows_b,
               sem_a, sem_b):
    wid = lax.axis_index("s") * _NC + lax.axis_index("c")
    base = wid * _BPW
    pltpu.sync_copy(idx_hbm.at[pl.ds(base, _BPW)], idx_v)
    bufs = (rows_a, rows_b)
    sems = (sem_a, sem_b)

    def _start(c):
        pltpu.async_copy(
            table_hbm.at[idx_v.at[pl.ds(c * _CH, _CH)]],
            bufs[c % 2], sems[c % 2])

    _start(0)
    for c in range(_NCH):
        if c + 1 < _NCH:
            _start(c + 1)
        pltpu.make_async_copy(
            table_hbm.at[idx_v.at[pl.ds(c * _CH, _CH)]],
            bufs[c % 2], sems[c % 2]).wait()
        pltpu.sync_copy(bufs[c % 2], out_hbm.at[pl.ds(base + c * _CH, _CH)])


def _loss_kernel(g_ref, w_ref, out_ref):
    i = pl.program_id(0)

    he = jnp.zeros((_BLK, _DP), dtype=jnp.float32)
    ho = jnp.zeros((_BLK, _DP), dtype=jnp.float32)
    for d in range(_DIM):
        gd = g_ref[d * _BLK:(d + 1) * _BLK, :]
        he = he + lax.bitcast_convert_type(gd << 16, jnp.float32)
        ho = ho + lax.bitcast_convert_type(gd & jnp.int32(-65536),
                                           jnp.float32)

    t = jnp.concatenate([jnp.tanh(he), jnp.tanh(ho)], axis=1)
    o = jnp.dot(t.astype(jnp.bfloat16), w_ref[...],
                preferred_element_type=jnp.float32)
    diff = o - 1.0

    row = i * _BLK + jax.lax.broadcasted_iota(jnp.int32, (_BLK, _D), 0)
    diff = jnp.where(row < _ROWS, diff, 0.0)
    s = jnp.sum(diff * diff, keepdims=True)

    @pl.when(i == 0)
    def _():
        out_ref[...] = jnp.zeros((1, 1), jnp.float32)

    out_ref[...] += s


def kernel(x, emb, W):
    xi = x[:, :-1].reshape(_ROWS, _DIM).astype(jnp.int32)
    idx = jnp.pad(xi, ((0, _NPOS - _ROWS), (0, 0)))
    idx = idx + jnp.arange(_DIM, dtype=jnp.int32) * _VOCAB
    idx = idx.reshape(_NBLK, _BLK, _DIM).transpose(0, 2, 1).reshape(_NIDX)

    bits = lax.bitcast_convert_type(emb.reshape(_DIM * _VOCAB, _D),
                                    jnp.int32)
    table = (lax.shift_right_logical(bits[:, :_DP], 16)
             | (bits[:, _DP:] & jnp.int32(-65536)))

    w_bf = W.astype(jnp.bfloat16)

    mesh = plsc.VectorSubcoreMesh(core_axis_name="c", subcore_axis_name="s")
    gather = functools.partial(
        pl.kernel,
        mesh=mesh,
        out_type=jax.ShapeDtypeStruct((_NIDX, _DP), jnp.int32),
        scratch_types=[
            pltpu.VMEM((_BPW,), jnp.int32),
            pltpu.VMEM((_CH, _DP), jnp.int32),
            pltpu.VMEM((_CH, _DP), jnp.int32),
            pltpu.SemaphoreType.DMA,
            pltpu.SemaphoreType.DMA,
        ],
    )(_sc_gather)
    g = gather(table, idx)

    out = pl.pallas_call(
        _loss_kernel,
        grid=(_NBLK,),
        in_specs=[
            pl.BlockSpec((_DIM * _BLK, _DP), lambda i: (i, 0)),
            pl.BlockSpec((_D, _D), lambda i: (0, 0)),
        ],
        out_specs=pl.BlockSpec((1, 1), lambda i: (0, 0)),
        out_shape=jax.ShapeDtypeStruct((1, 1), jnp.float32),
    )(g, w_bf)

    return out[0, 0] / (_ROWS * _D)


# speedup vs baseline: 3.6388x; 1.0004x over previous
import functools

import jax
import jax.numpy as jnp
from jax import lax
from jax.experimental import pallas as pl
from jax.experimental.pallas import tpu as pltpu
from jax.experimental.pallas import tpu_sc as plsc

_B, _S, _DIM = 4, 2048, 6
_VOCAB, _D = 512, 1024
_DP = _D // 2
_ROWS = _B * (_S - 1)
_BLK = 512
_NBLK = 16
_NPOS = _NBLK * _BLK
_NIDX = _NPOS * _DIM

_NC, _NS = 2, 16
_NW = _NC * _NS
_BPW = _NIDX // _NW
_CH = 96
_NCH = _BPW // _CH


def _sc_gather(table_hbm, idx_hbm, out_hbm, idx_v, rows_a, rows_b,
               sem_a, sem_b, wsem_a, wsem_b):
    wid = lax.axis_index("s") * _NC + lax.axis_index("c")
    base = wid * _BPW
    pltpu.sync_copy(idx_hbm.at[pl.ds(base, _BPW)], idx_v)
    bufs = (rows_a, rows_b)
    sems = (sem_a, sem_b)
    wsems = (wsem_a, wsem_b)

    def _start(c):
        pltpu.async_copy(
            table_hbm.at[idx_v.at[pl.ds(c * _CH, _CH)]],
            bufs[c % 2], sems[c % 2])

    def _wait_gather(c):
        pltpu.make_async_copy(
            table_hbm.at[idx_v.at[pl.ds(c * _CH, _CH)]],
            bufs[c % 2], sems[c % 2]).wait()

    def _start_write(c):
        pltpu.async_copy(
            bufs[c % 2], out_hbm.at[pl.ds(base + c * _CH, _CH)],
            wsems[c % 2])

    def _wait_write(c):
        pltpu.make_async_copy(
            bufs[c % 2], out_hbm.at[pl.ds(base + c * _CH, _CH)],
            wsems[c % 2]).wait()

    _start(0)
    _start(1)
    for c in range(_NCH):
        _wait_gather(c)
        _start_write(c)
        if c + 2 < _NCH:
            _wait_write(c)
            _start(c + 2)
    _wait_write(_NCH - 2)
    _wait_write(_NCH - 1)


def _loss_kernel(g_ref, w_ref, out_ref):
    i = pl.program_id(0)

    he = jnp.zeros((_BLK, _DP), dtype=jnp.float32)
    ho = jnp.zeros((_BLK, _DP), dtype=jnp.float32)
    for d in range(_DIM):
        gd = g_ref[d * _BLK:(d + 1) * _BLK, :]
        he = he + lax.bitcast_convert_type(gd << 16, jnp.float32)
        ho = ho + lax.bitcast_convert_type(gd & jnp.int32(-65536),
                                           jnp.float32)

    t = jnp.concatenate([jnp.tanh(he), jnp.tanh(ho)], axis=1)
    o = jnp.dot(t.astype(jnp.bfloat16), w_ref[...],
                preferred_element_type=jnp.float32)
    diff = o - 1.0

    row = i * _BLK + jax.lax.broadcasted_iota(jnp.int32, (_BLK, _D), 0)
    diff = jnp.where(row < _ROWS, diff, 0.0)
    s = jnp.sum(diff * diff, keepdims=True)

    @pl.when(i == 0)
    def _():
        out_ref[...] = jnp.zeros((1, 1), jnp.float32)

    out_ref[...] += s


def kernel(x, emb, W):
    xi = x[:, :-1].reshape(_ROWS, _DIM).astype(jnp.int32)
    idx = jnp.pad(xi, ((0, _NPOS - _ROWS), (0, 0)))
    idx = idx + jnp.arange(_DIM, dtype=jnp.int32) * _VOCAB
    idx = idx.reshape(_NBLK, _BLK, _DIM).transpose(0, 2, 1).reshape(_NIDX)

    bits = lax.bitcast_convert_type(emb.reshape(_DIM * _VOCAB, _D),
                                    jnp.int32)
    table = (lax.shift_right_logical(bits[:, :_DP], 16)
             | (bits[:, _DP:] & jnp.int32(-65536)))

    w_bf = W.astype(jnp.bfloat16)

    mesh = plsc.VectorSubcoreMesh(core_axis_name="c", subcore_axis_name="s")
    gather = functools.partial(
        pl.kernel,
        mesh=mesh,
        out_type=jax.ShapeDtypeStruct((_NIDX, _DP), jnp.int32),
        scratch_types=[
            pltpu.VMEM((_BPW,), jnp.int32),
            pltpu.VMEM((_CH, _DP), jnp.int32),
            pltpu.VMEM((_CH, _DP), jnp.int32),
            pltpu.SemaphoreType.DMA,
            pltpu.SemaphoreType.DMA,
            pltpu.SemaphoreType.DMA,
            pltpu.SemaphoreType.DMA,
        ],
    )(_sc_gather)
    g = gather(table, idx)

    out = pl.pallas_call(
        _loss_kernel,
        grid=(_NBLK,),
        in_specs=[
            pl.BlockSpec((_DIM * _BLK, _DP), lambda i: (i, 0)),
            pl.BlockSpec((_D, _D), lambda i: (0, 0)),
        ],
        out_specs=pl.BlockSpec((1, 1), lambda i: (0, 0)),
        out_shape=jax.ShapeDtypeStruct((1, 1), jnp.float32),
    )(g, w_bf)

    return out[0, 0] / (_ROWS * _D)


# speedup vs baseline: 4.0172x; 1.1040x over previous
import functools

import jax
import jax.numpy as jnp
from jax import lax
from jax.experimental import pallas as pl
from jax.experimental.pallas import tpu as pltpu
from jax.experimental.pallas import tpu_sc as plsc

_B, _S, _DIM = 4, 2048, 6
_VOCAB, _D = 512, 1024
_DP = _D // 2
_ROWS = _B * (_S - 1)
_BLK = 512
_NBLK = 16
_NPOS = _NBLK * _BLK
_RPB = _DIM * _BLK

_KOH = 3
_NSC = _NBLK - _KOH
_NIDX = _NSC * _RPB

_NC, _NS = 2, 16
_NW = _NC * _NS
_BPW = _NIDX // _NW
_CH = 96
_NCH = _BPW // _CH

assert _BPW * _NW == _NIDX and _NCH * _CH == _BPW


def _sc_gather(table_hbm, idx_hbm, out_hbm, idx_v, rows_a, rows_b,
               sem_a, sem_b, wsem_a, wsem_b):
    wid = lax.axis_index("s") * _NC + lax.axis_index("c")
    base = wid * _BPW
    pltpu.sync_copy(idx_hbm.at[pl.ds(base, _BPW)], idx_v)
    bufs = (rows_a, rows_b)
    sems = (sem_a, sem_b)
    wsems = (wsem_a, wsem_b)

    def _start(c):
        pltpu.async_copy(
            table_hbm.at[idx_v.at[pl.ds(c * _CH, _CH)]],
            bufs[c % 2], sems[c % 2])

    def _wait_gather(c):
        pltpu.make_async_copy(
            table_hbm.at[idx_v.at[pl.ds(c * _CH, _CH)]],
            bufs[c % 2], sems[c % 2]).wait()

    def _start_write(c):
        pltpu.async_copy(
            bufs[c % 2], out_hbm.at[pl.ds(base + c * _CH, _CH)],
            wsems[c % 2])

    def _wait_write(c):
        pltpu.make_async_copy(
            bufs[c % 2], out_hbm.at[pl.ds(base + c * _CH, _CH)],
            wsems[c % 2]).wait()

    _start(0)
    if _NCH > 1:
        _start(1)
    for c in range(_NCH):
        _wait_gather(c)
        _start_write(c)
        if c + 2 < _NCH:
            _wait_write(c)
            _start(c + 2)
    if _NCH > 1:
        _wait_write(_NCH - 2)
    _wait_write(_NCH - 1)


def _onehot_kernel(idx_ref, emb_ref, w_ref, out_ref):
    i = pl.program_id(0)

    h = jnp.zeros((_BLK, _D), dtype=jnp.float32)
    for d in range(_DIM):
        ids = idx_ref[0, d].reshape(_BLK, 1)
        oh = (jax.lax.broadcasted_iota(jnp.int32, (_BLK, _VOCAB), 1)
              == ids).astype(jnp.bfloat16)
        h = h + jnp.dot(oh, emb_ref[d], preferred_element_type=jnp.float32)

    t = jnp.tanh(h).astype(jnp.bfloat16)
    o = jnp.dot(t, w_ref[...], preferred_element_type=jnp.float32)
    diff = o - 1.0
    s = jnp.sum(diff * diff, keepdims=True)

    @pl.when(i == 0)
    def _():
        out_ref[...] = jnp.zeros((1, 1), jnp.float32)

    out_ref[...] += s


def _consume_kernel(g_ref, w_ref, out_ref):
    i = pl.program_id(0)

    he = jnp.zeros((_BLK, _DP), dtype=jnp.float32)
    ho = jnp.zeros((_BLK, _DP), dtype=jnp.float32)
    for d in range(_DIM):
        gd = g_ref[d * _BLK:(d + 1) * _BLK, :]
        he = he + lax.bitcast_convert_type(gd << 16, jnp.float32)
        ho = ho + lax.bitcast_convert_type(gd & jnp.int32(-65536),
                                           jnp.float32)

    t = jnp.concatenate([jnp.tanh(he), jnp.tanh(ho)], axis=1)
    o = jnp.dot(t.astype(jnp.bfloat16), w_ref[...],
                preferred_element_type=jnp.float32)
    diff = o - 1.0

    row = (_KOH + i) * _BLK + jax.lax.broadcasted_iota(
        jnp.int32, (_BLK, _D), 0)
    diff = jnp.where(row < _ROWS, diff, 0.0)
    s = jnp.sum(diff * diff, keepdims=True)

    @pl.when(i == 0)
    def _():
        out_ref[...] = jnp.zeros((1, 1), jnp.float32)

    out_ref[...] += s


def kernel(x, emb, W):
    xi = x[:, :-1].reshape(_ROWS, _DIM).astype(jnp.int32)
    idx = jnp.pad(xi, ((0, _NPOS - _ROWS), (0, 0)))
    idx3 = (idx.reshape(_NBLK, _BLK, _DIM)
            .transpose(0, 2, 1))
    offs = (jnp.arange(_DIM, dtype=jnp.int32) * _VOCAB)[None, :, None]
    idx_sc = (idx3[_KOH:] + offs).reshape(_NIDX)

    bits = lax.bitcast_convert_type(emb.reshape(_DIM * _VOCAB, _D),
                                    jnp.int32)
    table = (lax.shift_right_logical(bits[:, :_DP], 16)
             | (bits[:, _DP:] & jnp.int32(-65536)))

    emb_bf = emb.astype(jnp.bfloat16)
    w_bf = W.astype(jnp.bfloat16)

    mesh = plsc.VectorSubcoreMesh(core_axis_name="c", subcore_axis_name="s")
    gather = functools.partial(
        pl.kernel,
        mesh=mesh,
        out_type=jax.ShapeDtypeStruct((_NIDX, _DP), jnp.int32),
        scratch_types=[
            pltpu.VMEM((_BPW,), jnp.int32),
            pltpu.VMEM((_CH, _DP), jnp.int32),
            pltpu.VMEM((_CH, _DP), jnp.int32),
            pltpu.SemaphoreType.DMA,
            pltpu.SemaphoreType.DMA,
            pltpu.SemaphoreType.DMA,
            pltpu.SemaphoreType.DMA,
        ],
    )(_sc_gather)
    g = gather(table, idx_sc)

    s_oh = pl.pallas_call(
        _onehot_kernel,
        grid=(_KOH,),
        in_specs=[
            pl.BlockSpec((1, _DIM, _BLK), lambda i: (i, 0, 0)),
            pl.BlockSpec((_DIM, _VOCAB, _D), lambda i: (0, 0, 0)),
            pl.BlockSpec((_D, _D), lambda i: (0, 0)),
        ],
        out_specs=pl.BlockSpec((1, 1), lambda i: (0, 0)),
        out_shape=jax.ShapeDtypeStruct((1, 1), jnp.float32),
    )(idx3[:_KOH], emb_bf, w_bf)

    s_sc = pl.pallas_call(
        _consume_kernel,
        grid=(_NSC,),
        in_specs=[
            pl.BlockSpec((_RPB, _DP), lambda i: (i, 0)),
            pl.BlockSpec((_D, _D), lambda i: (0, 0)),
        ],
        out_specs=pl.BlockSpec((1, 1), lambda i: (0, 0)),
        out_shape=jax.ShapeDtypeStruct((1, 1), jnp.float32),
    )(g, w_bf)

    return (s_oh[0, 0] + s_sc[0, 0]) / (_ROWS * _D)


# speedup vs baseline: 4.8863x; 1.2164x over previous
import functools

import jax
import jax.numpy as jnp
from jax import lax
from jax.experimental import pallas as pl
from jax.experimental.pallas import tpu as pltpu
from jax.experimental.pallas import tpu_sc as plsc

_B, _S, _DIM = 4, 2048, 6
_VOCAB, _D = 512, 1024
_DP = _D // 2
_ROWS = _B * (_S - 1)
_BLK = 512
_NBLK = 16
_NPOS = _NBLK * _BLK
_RPB = _DIM * _BLK

_KOH = 8
_NSC = _NBLK - _KOH
_NIDX = _NSC * _RPB

_NC, _NS = 2, 16
_NW = _NC * _NS
_BPW = _NIDX // _NW
_CH = 96
_NCH = _BPW // _CH

assert _BPW * _NW == _NIDX and _NCH * _CH == _BPW


def _sc_gather(table_hbm, idx_hbm, out_hbm, idx_v, rows_a, rows_b,
               sem_a, sem_b, wsem_a, wsem_b):
    wid = lax.axis_index("s") * _NC + lax.axis_index("c")
    base = wid * _BPW
    pltpu.sync_copy(idx_hbm.at[pl.ds(base, _BPW)], idx_v)
    bufs = (rows_a, rows_b)
    sems = (sem_a, sem_b)
    wsems = (wsem_a, wsem_b)

    def _start(c):
        pltpu.async_copy(
            table_hbm.at[idx_v.at[pl.ds(c * _CH, _CH)]],
            bufs[c % 2], sems[c % 2])

    def _wait_gather(c):
        pltpu.make_async_copy(
            table_hbm.at[idx_v.at[pl.ds(c * _CH, _CH)]],
            bufs[c % 2], sems[c % 2]).wait()

    def _start_write(c):
        pltpu.async_copy(
            bufs[c % 2], out_hbm.at[pl.ds(base + c * _CH, _CH)],
            wsems[c % 2])

    def _wait_write(c):
        pltpu.make_async_copy(
            bufs[c % 2], out_hbm.at[pl.ds(base + c * _CH, _CH)],
            wsems[c % 2]).wait()

    _start(0)
    if _NCH > 1:
        _start(1)
    for c in range(_NCH):
        _wait_gather(c)
        _start_write(c)
        if c + 2 < _NCH:
            _wait_write(c)
            _start(c + 2)
    if _NCH > 1:
        _wait_write(_NCH - 2)
    _wait_write(_NCH - 1)


def _onehot_kernel(idx_ref, emb_ref, w_ref, out_ref):
    i = pl.program_id(0)

    h = jnp.zeros((_BLK, _D), dtype=jnp.float32)
    for d in range(_DIM):
        ids = idx_ref[0, d].reshape(_BLK, 1)
        oh = (jax.lax.broadcasted_iota(jnp.int32, (_BLK, _VOCAB), 1)
              == ids).astype(jnp.float8_e4m3fn)
        h = h + jnp.dot(oh, emb_ref[d], preferred_element_type=jnp.float32)

    t = (jnp.tanh(h * 0.125) * 8.0).astype(jnp.float8_e4m3fn)
    o = jnp.dot(t, w_ref[...], preferred_element_type=jnp.float32) * (1.0 / 128.0)
    diff = o - 1.0
    s = jnp.sum(diff * diff, keepdims=True)

    @pl.when(i == 0)
    def _():
        out_ref[...] = jnp.zeros((1, 1), jnp.float32)

    out_ref[...] += s


def _consume_kernel(g_ref, w_ref, out_ref):
    i = pl.program_id(0)

    he = jnp.zeros((_BLK, _DP), dtype=jnp.float32)
    ho = jnp.zeros((_BLK, _DP), dtype=jnp.float32)
    for d in range(_DIM):
        gd = g_ref[d * _BLK:(d + 1) * _BLK, :]
        he = he + lax.bitcast_convert_type(gd << 16, jnp.float32)
        ho = ho + lax.bitcast_convert_type(gd & jnp.int32(-65536),
                                           jnp.float32)

    t = jnp.concatenate([jnp.tanh(he), jnp.tanh(ho)], axis=1)
    o = jnp.dot(t.astype(jnp.bfloat16), w_ref[...],
                preferred_element_type=jnp.float32)
    diff = o - 1.0

    row = (_KOH + i) * _BLK + jax.lax.broadcasted_iota(
        jnp.int32, (_BLK, _D), 0)
    diff = jnp.where(row < _ROWS, diff, 0.0)
    s = jnp.sum(diff * diff, keepdims=True)

    @pl.when(i == 0)
    def _():
        out_ref[...] = jnp.zeros((1, 1), jnp.float32)

    out_ref[...] += s


def kernel(x, emb, W):
    xi = x[:, :-1].reshape(_ROWS, _DIM).astype(jnp.int32)
    idx = jnp.pad(xi, ((0, _NPOS - _ROWS), (0, 0)))
    idx3 = (idx.reshape(_NBLK, _BLK, _DIM)
            .transpose(0, 2, 1))
    offs = (jnp.arange(_DIM, dtype=jnp.int32) * _VOCAB)[None, :, None]
    idx_sc = (idx3[_KOH:] + offs).reshape(_NIDX)

    bits = lax.bitcast_convert_type(emb.reshape(_DIM * _VOCAB, _D),
                                    jnp.int32)
    table = (lax.shift_right_logical(bits[:, :_DP], 16)
             | (bits[:, _DP:] & jnp.int32(-65536)))

    emb_f8 = (emb * 8.0).astype(jnp.float8_e4m3fn)
    w_f8 = (W * 16.0).astype(jnp.float8_e4m3fn)
    w_bf = W.astype(jnp.bfloat16)

    mesh = plsc.VectorSubcoreMesh(core_axis_name="c", subcore_axis_name="s")
    gather = functools.partial(
        pl.kernel,
        mesh=mesh,
        out_type=jax.ShapeDtypeStruct((_NIDX, _DP), jnp.int32),
        scratch_types=[
            pltpu.VMEM((_BPW,), jnp.int32),
            pltpu.VMEM((_CH, _DP), jnp.int32),
            pltpu.VMEM((_CH, _DP), jnp.int32),
            pltpu.SemaphoreType.DMA,
            pltpu.SemaphoreType.DMA,
            pltpu.SemaphoreType.DMA,
            pltpu.SemaphoreType.DMA,
        ],
    )(_sc_gather)
    g = gather(table, idx_sc)

    s_oh = pl.pallas_call(
        _onehot_kernel,
        grid=(_KOH,),
        in_specs=[
            pl.BlockSpec((1, _DIM, _BLK), lambda i: (i, 0, 0)),
            pl.BlockSpec((_DIM, _VOCAB, _D), lambda i: (0, 0, 0)),
            pl.BlockSpec((_D, _D), lambda i: (0, 0)),
        ],
        out_specs=pl.BlockSpec((1, 1), lambda i: (0, 0)),
        out_shape=jax.ShapeDtypeStruct((1, 1), jnp.float32),
    )(idx3[:_KOH], emb_f8, w_f8)

    s_sc = pl.pallas_call(
        _consume_kernel,
        grid=(_NSC,),
        in_specs=[
            pl.BlockSpec((_RPB, _DP), lambda i: (i, 0)),
            pl.BlockSpec((_D, _D), lambda i: (0, 0)),
        ],
        out_specs=pl.BlockSpec((1, 1), lambda i: (0, 0)),
        out_shape=jax.ShapeDtypeStruct((1, 1), jnp.float32),
    )(g, w_bf)

    return (s_oh[0, 0] + s_sc[0, 0]) / (_ROWS * _D)


# speedup vs baseline: 6.1173x; 1.2519x over previous
import functools

import jax
import jax.numpy as jnp
from jax import lax
from jax.experimental import pallas as pl
from jax.experimental.pallas import tpu as pltpu
from jax.experimental.pallas import tpu_sc as plsc

_B, _S, _DIM = 4, 2048, 6
_VOCAB, _D = 512, 1024
_DP = _D // 2
_ROWS = _B * (_S - 1)
_BLK = 512
_NBLK = 16
_NPOS = _NBLK * _BLK
_RPB = _DIM * _BLK

_KOH = 8
_NSC = _NBLK - _KOH
_NIDX = _NSC * _RPB

_NC, _NS = 2, 16
_NW = _NC * _NS
_BPW = _NIDX // _NW
_CH = 96
_NCH = _BPW // _CH

assert _BPW * _NW == _NIDX and _NCH * _CH == _BPW


def _sc_gather(table_hbm, idx_hbm, out_hbm, idx_v, rows_a, rows_b,
               sem_a, sem_b, wsem_a, wsem_b):
    wid = lax.axis_index("s") * _NC + lax.axis_index("c")
    base = wid * _BPW
    pltpu.sync_copy(idx_hbm.at[pl.ds(base, _BPW)], idx_v)
    bufs = (rows_a, rows_b)
    sems = (sem_a, sem_b)
    wsems = (wsem_a, wsem_b)

    def _start(c):
        pltpu.async_copy(
            table_hbm.at[idx_v.at[pl.ds(c * _CH, _CH)]],
            bufs[c % 2], sems[c % 2])

    def _wait_gather(c):
        pltpu.make_async_copy(
            table_hbm.at[idx_v.at[pl.ds(c * _CH, _CH)]],
            bufs[c % 2], sems[c % 2]).wait()

    def _start_write(c):
        pltpu.async_copy(
            bufs[c % 2], out_hbm.at[pl.ds(base + c * _CH, _CH)],
            wsems[c % 2])

    def _wait_write(c):
        pltpu.make_async_copy(
            bufs[c % 2], out_hbm.at[pl.ds(base + c * _CH, _CH)],
            wsems[c % 2]).wait()

    _start(0)
    if _NCH > 1:
        _start(1)
    for c in range(_NCH):
        _wait_gather(c)
        _start_write(c)
        if c + 2 < _NCH:
            _wait_write(c)
            _start(c + 2)
    if _NCH > 1:
        _wait_write(_NCH - 2)
    _wait_write(_NCH - 1)


def _prep_kernel(emb_ref, pack_ref, f8_ref):
    e = emb_ref[0]
    bits = lax.bitcast_convert_type(e, jnp.int32)
    pack_ref[...] = (lax.shift_right_logical(bits[:, :_DP], 16)
                     | (bits[:, _DP:] & jnp.int32(-65536)))
    f8_ref[0] = (e * 8.0).astype(jnp.float8_e4m3fn)


def _onehot_kernel(idx_ref, emb_ref, w_ref, out_ref):
    i = pl.program_id(0)

    h = jnp.zeros((_BLK, _D), dtype=jnp.float32)
    for d in range(_DIM):
        ids = idx_ref[0, d].reshape(_BLK, 1)
        oh = (jax.lax.broadcasted_iota(jnp.int32, (_BLK, _VOCAB), 1)
              == ids).astype(jnp.float8_e4m3fn)
        h = h + jnp.dot(oh, emb_ref[d], preferred_element_type=jnp.float32)

    t = (jnp.tanh(h * 0.125) * 8.0).astype(jnp.float8_e4m3fn)
    o = jnp.dot(t, w_ref[...], preferred_element_type=jnp.float32) * (1.0 / 128.0)
    diff = o - 1.0
    s = jnp.sum(diff * diff, keepdims=True)

    @pl.when(i == 0)
    def _():
        out_ref[...] = jnp.zeros((1, 1), jnp.float32)

    out_ref[...] += s


def _consume_kernel(g_ref, w_ref, out_ref):
    i = pl.program_id(0)

    he = jnp.zeros((_BLK, _DP), dtype=jnp.float32)
    ho = jnp.zeros((_BLK, _DP), dtype=jnp.float32)
    for d in range(_DIM):
        gd = g_ref[d * _BLK:(d + 1) * _BLK, :]
        he = he + lax.bitcast_convert_type(gd << 16, jnp.float32)
        ho = ho + lax.bitcast_convert_type(gd & jnp.int32(-65536),
                                           jnp.float32)

    t = jnp.concatenate([jnp.tanh(he), jnp.tanh(ho)], axis=1)
    o = jnp.dot(t.astype(jnp.bfloat16), w_ref[...],
                preferred_element_type=jnp.float32)
    diff = o - 1.0

    row = (_KOH + i) * _BLK + jax.lax.broadcasted_iota(
        jnp.int32, (_BLK, _D), 0)
    diff = jnp.where(row < _ROWS, diff, 0.0)
    s = jnp.sum(diff * diff, keepdims=True)

    @pl.when(i == 0)
    def _():
        out_ref[...] = jnp.zeros((1, 1), jnp.float32)

    out_ref[...] += s


def kernel(x, emb, W):
    xi = x[:, :-1].reshape(_ROWS, _DIM).astype(jnp.int32)
    idx = jnp.pad(xi, ((0, _NPOS - _ROWS), (0, 0)))
    idx3 = (idx.reshape(_NBLK, _BLK, _DIM)
            .transpose(0, 2, 1))
    offs = (jnp.arange(_DIM, dtype=jnp.int32) * _VOCAB)[None, :, None]
    idx_sc = (idx3[_KOH:] + offs).reshape(_NIDX)

    table, emb_f8 = pl.pallas_call(
        _prep_kernel,
        grid=(_DIM,),
        in_specs=[pl.BlockSpec((1, _VOCAB, _D), lambda i: (i, 0, 0))],
        out_specs=[
            pl.BlockSpec((_VOCAB, _DP), lambda i: (i, 0)),
            pl.BlockSpec((1, _VOCAB, _D), lambda i: (i, 0, 0)),
        ],
        out_shape=[
            jax.ShapeDtypeStruct((_DIM * _VOCAB, _DP), jnp.int32),
            jax.ShapeDtypeStruct((_DIM, _VOCAB, _D), jnp.float8_e4m3fn),
        ],
    )(emb)

    w_f8 = (W * 16.0).astype(jnp.float8_e4m3fn)
    w_bf = W.astype(jnp.bfloat16)

    mesh = plsc.VectorSubcoreMesh(core_axis_name="c", subcore_axis_name="s")
    gather = functools.partial(
        pl.kernel,
        mesh=mesh,
        out_type=jax.ShapeDtypeStruct((_NIDX, _DP), jnp.int32),
        scratch_types=[
            pltpu.VMEM((_BPW,), jnp.int32),
            pltpu.VMEM((_CH, _DP), jnp.int32),
            pltpu.VMEM((_CH, _DP), jnp.int32),
            pltpu.SemaphoreType.DMA,
            pltpu.SemaphoreType.DMA,
            pltpu.SemaphoreType.DMA,
            pltpu.SemaphoreType.DMA,
        ],
    )(_sc_gather)
    g = gather(table, idx_sc)

    s_oh = pl.pallas_call(
        _onehot_kernel,
        grid=(_KOH,),
        in_specs=[
            pl.BlockSpec((1, _DIM, _BLK), lambda i: (i, 0, 0)),
            pl.BlockSpec((_DIM, _VOCAB, _D), lambda i: (0, 0, 0)),
            pl.BlockSpec((_D, _D), lambda i: (0, 0)),
        ],
        out_specs=pl.BlockSpec((1, 1), lambda i: (0, 0)),
        out_shape=jax.ShapeDtypeStruct((1, 1), jnp.float32),
    )(idx3[:_KOH], emb_f8, w_f8)

    s_sc = pl.pallas_call(
        _consume_kernel,
        grid=(_NSC,),
        in_specs=[
            pl.BlockSpec((_RPB, _DP), lambda i: (i, 0)),
            pl.BlockSpec((_D, _D), lambda i: (0, 0)),
        ],
        out_specs=pl.BlockSpec((1, 1), lambda i: (0, 0)),
        out_shape=jax.ShapeDtypeStruct((1, 1), jnp.float32),
    )(g, w_bf)

    return (s_oh[0, 0] + s_sc[0, 0]) / (_ROWS * _D)


# speedup vs baseline: 6.6141x; 1.0812x over previous
import functools

import jax
import jax.numpy as jnp
from jax import lax
from jax.experimental import pallas as pl
from jax.experimental.pallas import tpu as pltpu
from jax.experimental.pallas import tpu_sc as plsc

_B, _S, _DIM = 4, 2048, 6
_VOCAB, _D = 512, 1024
_DP = _D // 2
_ROWS = _B * (_S - 1)
_BLK = 512
_NBLK = 16
_NPOS = _NBLK * _BLK
_RPB = _DIM * _BLK

_KOH = 9
_NSC = _NBLK - _KOH
_NIDX = _NSC * _RPB

_NC, _NS = 2, 16
_NW = _NC * _NS
_BPW = _NIDX // _NW
_CH = 96
_NCH = _BPW // _CH

assert _BPW * _NW == _NIDX and _NCH * _CH == _BPW


def _sc_gather(table_hbm, idx_hbm, out_hbm, idx_v, rows_a, rows_b,
               sem_a, sem_b, wsem_a, wsem_b):
    wid = lax.axis_index("s") * _NC + lax.axis_index("c")
    base = wid * _BPW
    pltpu.sync_copy(idx_hbm.at[pl.ds(base, _BPW)], idx_v)
    bufs = (rows_a, rows_b)
    sems = (sem_a, sem_b)
    wsems = (wsem_a, wsem_b)

    def _start(c):
        pltpu.async_copy(
            table_hbm.at[idx_v.at[pl.ds(c * _CH, _CH)]],
            bufs[c % 2], sems[c % 2])

    def _wait_gather(c):
        pltpu.make_async_copy(
            table_hbm.at[idx_v.at[pl.ds(c * _CH, _CH)]],
            bufs[c % 2], sems[c % 2]).wait()

    def _start_write(c):
        pltpu.async_copy(
            bufs[c % 2], out_hbm.at[pl.ds(base + c * _CH, _CH)],
            wsems[c % 2])

    def _wait_write(c):
        pltpu.make_async_copy(
            bufs[c % 2], out_hbm.at[pl.ds(base + c * _CH, _CH)],
            wsems[c % 2]).wait()

    _start(0)
    if _NCH > 1:
        _start(1)
    for c in range(_NCH):
        _wait_gather(c)
        _start_write(c)
        if c + 2 < _NCH:
            _wait_write(c)
            _start(c + 2)
    if _NCH > 1:
        _wait_write(_NCH - 2)
    _wait_write(_NCH - 1)


def _prep_kernel(emb_ref, pack_ref, f8_ref):
    e = emb_ref[0]
    bits = lax.bitcast_convert_type(e, jnp.int32)
    pack_ref[...] = (lax.shift_right_logical(bits[:, :_DP], 16)
                     | (bits[:, _DP:] & jnp.int32(-65536)))
    f8_ref[0] = (e * 8.0).astype(jnp.float8_e4m3fn)


def _onehot_kernel(idx_ref, emb_ref, w_ref, out_ref):
    i = pl.program_id(0)

    h = jnp.zeros((_BLK, _D), dtype=jnp.float32)
    for d in range(_DIM):
        ids = idx_ref[0, d].reshape(_BLK, 1)
        oh = (jax.lax.broadcasted_iota(jnp.int32, (_BLK, _VOCAB), 1)
              == ids).astype(jnp.float8_e4m3fn)
        h = h + jnp.dot(oh, emb_ref[d], preferred_element_type=jnp.float32)

    t = (jnp.tanh(h * 0.125) * 8.0).astype(jnp.float8_e4m3fn)
    o = jnp.dot(t, w_ref[...], preferred_element_type=jnp.float32) * (1.0 / 128.0)
    diff = o - 1.0
    s = jnp.sum(diff * diff, keepdims=True)

    @pl.when(i == 0)
    def _():
        out_ref[...] = jnp.zeros((1, 1), jnp.float32)

    out_ref[...] += s


def _consume_kernel(g_ref, w_ref, out_ref):
    i = pl.program_id(0)

    he = jnp.zeros((_BLK, _DP), dtype=jnp.float32)
    ho = jnp.zeros((_BLK, _DP), dtype=jnp.float32)
    for d in range(_DIM):
        gd = g_ref[d * _BLK:(d + 1) * _BLK, :]
        he = he + lax.bitcast_convert_type(gd << 16, jnp.float32)
        ho = ho + lax.bitcast_convert_type(gd & jnp.int32(-65536),
                                           jnp.float32)

    t = jnp.concatenate([jnp.tanh(he), jnp.tanh(ho)], axis=1)
    o = jnp.dot(t.astype(jnp.bfloat16), w_ref[...],
                preferred_element_type=jnp.float32)
    diff = o - 1.0

    row = (_KOH + i) * _BLK + jax.lax.broadcasted_iota(
        jnp.int32, (_BLK, _D), 0)
    diff = jnp.where(row < _ROWS, diff, 0.0)
    s = jnp.sum(diff * diff, keepdims=True)

    @pl.when(i == 0)
    def _():
        out_ref[...] = jnp.zeros((1, 1), jnp.float32)

    out_ref[...] += s


def kernel(x, emb, W):
    xi = x[:, :-1].reshape(_ROWS, _DIM).astype(jnp.int32)
    idx = jnp.pad(xi, ((0, _NPOS - _ROWS), (0, 0)))
    idx3 = (idx.reshape(_NBLK, _BLK, _DIM)
            .transpose(0, 2, 1))
    offs = (jnp.arange(_DIM, dtype=jnp.int32) * _VOCAB)[None, :, None]
    idx_sc = (idx3[_KOH:] + offs).reshape(_NIDX)

    table, emb_f8 = pl.pallas_call(
        _prep_kernel,
        grid=(_DIM,),
        in_specs=[pl.BlockSpec((1, _VOCAB, _D), lambda i: (i, 0, 0))],
        out_specs=[
            pl.BlockSpec((_VOCAB, _DP), lambda i: (i, 0)),
            pl.BlockSpec((1, _VOCAB, _D), lambda i: (i, 0, 0)),
        ],
        out_shape=[
            jax.ShapeDtypeStruct((_DIM * _VOCAB, _DP), jnp.int32),
            jax.ShapeDtypeStruct((_DIM, _VOCAB, _D), jnp.float8_e4m3fn),
        ],
    )(emb)

    w_f8 = (W * 16.0).astype(jnp.float8_e4m3fn)
    w_bf = W.astype(jnp.bfloat16)

    mesh = plsc.VectorSubcoreMesh(core_axis_name="c", subcore_axis_name="s")
    gather = functools.partial(
        pl.kernel,
        mesh=mesh,
        out_type=jax.ShapeDtypeStruct((_NIDX, _DP), jnp.int32),
        scratch_types=[
            pltpu.VMEM((_BPW,), jnp.int32),
            pltpu.VMEM((_CH, _DP), jnp.int32),
            pltpu.VMEM((_CH, _DP), jnp.int32),
            pltpu.SemaphoreType.DMA,
            pltpu.SemaphoreType.DMA,
            pltpu.SemaphoreType.DMA,
            pltpu.SemaphoreType.DMA,
        ],
    )(_sc_gather)
    g = gather(table, idx_sc)

    s_oh = pl.pallas_call(
        _onehot_kernel,
        grid=(_KOH,),
        in_specs=[
            pl.BlockSpec((1, _DIM, _BLK), lambda i: (i, 0, 0)),
            pl.BlockSpec((_DIM, _VOCAB, _D), lambda i: (0, 0, 0)),
            pl.BlockSpec((_D, _D), lambda i: (0, 0)),
        ],
        out_specs=pl.BlockSpec((1, 1), lambda i: (0, 0)),
        out_shape=jax.ShapeDtypeStruct((1, 1), jnp.float32),
    )(idx3[:_KOH], emb_f8, w_f8)

    s_sc = pl.pallas_call(
        _consume_kernel,
        grid=(_NSC,),
        in_specs=[
            pl.BlockSpec((_RPB, _DP), lambda i: (i, 0)),
            pl.BlockSpec((_D, _D), lambda i: (0, 0)),
        ],
        out_specs=pl.BlockSpec((1, 1), lambda i: (0, 0)),
        out_shape=jax.ShapeDtypeStruct((1, 1), jnp.float32),
    )(g, w_bf)

    return (s_oh[0, 0] + s_sc[0, 0]) / (_ROWS * _D)


# speedup vs baseline: 6.7144x; 1.0152x over previous
import functools

import jax
import jax.numpy as jnp
from jax import lax
from jax.experimental import pallas as pl
from jax.experimental.pallas import tpu as pltpu
from jax.experimental.pallas import tpu_sc as plsc

_B, _S, _DIM = 4, 2048, 6
_VOCAB, _D = 512, 1024
_DP = _D // 2
_ROWS = _B * (_S - 1)
_BLK = 512
_NBLK = 16
_NPOS = _NBLK * _BLK
_RPB = _DIM * _BLK

_KOH = 10
_NSC = _NBLK - _KOH
_NIDX = _NSC * _RPB

_NC, _NS = 2, 16
_NW = _NC * _NS
_BPW = _NIDX // _NW
_CH = 96
_NCH = _BPW // _CH

assert _BPW * _NW == _NIDX and _NCH * _CH == _BPW


def _sc_gather(table_hbm, idx_hbm, out_hbm, idx_v, rows_a, rows_b,
               sem_a, sem_b, wsem_a, wsem_b):
    wid = lax.axis_index("s") * _NC + lax.axis_index("c")
    base = wid * _BPW
    pltpu.sync_copy(idx_hbm.at[pl.ds(base, _BPW)], idx_v)
    bufs = (rows_a, rows_b)
    sems = (sem_a, sem_b)
    wsems = (wsem_a, wsem_b)

    def _start(c):
        pltpu.async_copy(
            table_hbm.at[idx_v.at[pl.ds(c * _CH, _CH)]],
            bufs[c % 2], sems[c % 2])

    def _wait_gather(c):
        pltpu.make_async_copy(
            table_hbm.at[idx_v.at[pl.ds(c * _CH, _CH)]],
            bufs[c % 2], sems[c % 2]).wait()

    def _start_write(c):
        pltpu.async_copy(
            bufs[c % 2], out_hbm.at[pl.ds(base + c * _CH, _CH)],
            wsems[c % 2])

    def _wait_write(c):
        pltpu.make_async_copy(
            bufs[c % 2], out_hbm.at[pl.ds(base + c * _CH, _CH)],
            wsems[c % 2]).wait()

    _start(0)
    if _NCH > 1:
        _start(1)
    for c in range(_NCH):
        _wait_gather(c)
        _start_write(c)
        if c + 2 < _NCH:
            _wait_write(c)
            _start(c + 2)
    if _NCH > 1:
        _wait_write(_NCH - 2)
    _wait_write(_NCH - 1)


def _prep_kernel(emb_ref, pack_ref, f8_ref):
    e = emb_ref[0]
    bits = lax.bitcast_convert_type(e, jnp.int32)
    pack_ref[...] = (lax.shift_right_logical(bits[:, :_DP], 16)
                     | (bits[:, _DP:] & jnp.int32(-65536)))
    f8_ref[0] = (e * 8.0).astype(jnp.float8_e4m3fn)


def _onehot_kernel(idx_ref, emb_ref, w_ref, out_ref):
    i = pl.program_id(0)

    h = jnp.zeros((_BLK, _D), dtype=jnp.float32)
    for d in range(_DIM):
        ids = idx_ref[0, d].reshape(_BLK, 1)
        oh = (jax.lax.broadcasted_iota(jnp.int32, (_BLK, _VOCAB), 1)
              == ids).astype(jnp.float8_e4m3fn)
        h = h + jnp.dot(oh, emb_ref[d], preferred_element_type=jnp.float32)

    t = (jnp.tanh(h * 0.125) * 8.0).astype(jnp.float8_e4m3fn)
    o = jnp.dot(t, w_ref[...], preferred_element_type=jnp.float32) * (1.0 / 128.0)
    diff = o - 1.0
    s = jnp.sum(diff * diff, keepdims=True)

    @pl.when(i == 0)
    def _():
        out_ref[...] = jnp.zeros((1, 1), jnp.float32)

    out_ref[...] += s


def _consume_kernel(g_ref, w_ref, out_ref):
    i = pl.program_id(0)

    he = jnp.zeros((_BLK, _DP), dtype=jnp.float32)
    ho = jnp.zeros((_BLK, _DP), dtype=jnp.float32)
    for d in range(_DIM):
        gd = g_ref[d * _BLK:(d + 1) * _BLK, :]
        he = he + lax.bitcast_convert_type(gd << 16, jnp.float32)
        ho = ho + lax.bitcast_convert_type(gd & jnp.int32(-65536),
                                           jnp.float32)

    t = jnp.concatenate([jnp.tanh(he), jnp.tanh(ho)], axis=1)
    o = jnp.dot(t.astype(jnp.bfloat16), w_ref[...],
                preferred_element_type=jnp.float32)
    diff = o - 1.0

    row = (_KOH + i) * _BLK + jax.lax.broadcasted_iota(
        jnp.int32, (_BLK, _D), 0)
    diff = jnp.where(row < _ROWS, diff, 0.0)
    s = jnp.sum(diff * diff, keepdims=True)

    @pl.when(i == 0)
    def _():
        out_ref[...] = jnp.zeros((1, 1), jnp.float32)

    out_ref[...] += s


def kernel(x, emb, W):
    xi = x[:, :-1].reshape(_ROWS, _DIM).astype(jnp.int32)
    idx = jnp.pad(xi, ((0, _NPOS - _ROWS), (0, 0)))
    idx3 = (idx.reshape(_NBLK, _BLK, _DIM)
            .transpose(0, 2, 1))
    offs = (jnp.arange(_DIM, dtype=jnp.int32) * _VOCAB)[None, :, None]
    idx_sc = (idx3[_KOH:] + offs).reshape(_NIDX)

    table, emb_f8 = pl.pallas_call(
        _prep_kernel,
        grid=(_DIM,),
        in_specs=[pl.BlockSpec((1, _VOCAB, _D), lambda i: (i, 0, 0))],
        out_specs=[
            pl.BlockSpec((_VOCAB, _DP), lambda i: (i, 0)),
            pl.BlockSpec((1, _VOCAB, _D), lambda i: (i, 0, 0)),
        ],
        out_shape=[
            jax.ShapeDtypeStruct((_DIM * _VOCAB, _DP), jnp.int32),
            jax.ShapeDtypeStruct((_DIM, _VOCAB, _D), jnp.float8_e4m3fn),
        ],
    )(emb)

    w_f8 = (W * 16.0).astype(jnp.float8_e4m3fn)
    w_bf = W.astype(jnp.bfloat16)

    mesh = plsc.VectorSubcoreMesh(core_axis_name="c", subcore_axis_name="s")
    gather = functools.partial(
        pl.kernel,
        mesh=mesh,
        out_type=jax.ShapeDtypeStruct((_NIDX, _DP), jnp.int32),
        scratch_types=[
            pltpu.VMEM((_BPW,), jnp.int32),
            pltpu.VMEM((_CH, _DP), jnp.int32),
            pltpu.VMEM((_CH, _DP), jnp.int32),
            pltpu.SemaphoreType.DMA,
            pltpu.SemaphoreType.DMA,
            pltpu.SemaphoreType.DMA,
            pltpu.SemaphoreType.DMA,
        ],
    )(_sc_gather)
    g = gather(table, idx_sc)

    s_oh = pl.pallas_call(
        _onehot_kernel,
        grid=(_KOH,),
        in_specs=[
            pl.BlockSpec((1, _DIM, _BLK), lambda i: (i, 0, 0)),
            pl.BlockSpec((_DIM, _VOCAB, _D), lambda i: (0, 0, 0)),
            pl.BlockSpec((_D, _D), lambda i: (0, 0)),
        ],
        out_specs=pl.BlockSpec((1, 1), lambda i: (0, 0)),
        out_shape=jax.ShapeDtypeStruct((1, 1), jnp.float32),
    )(idx3[:_KOH], emb_f8, w_f8)

    s_sc = pl.pallas_call(
        _consume_kernel,
        grid=(_NSC,),
        in_specs=[
            pl.BlockSpec((_RPB, _DP), lambda i: (i, 0)),
            pl.BlockSpec((_D, _D), lambda i: (0, 0)),
        ],
        out_specs=pl.BlockSpec((1, 1), lambda i: (0, 0)),
        out_shape=jax.ShapeDtypeStruct((1, 1), jnp.float32),
    )(g, w_bf)

    return (s_oh[0, 0] + s_sc[0, 0]) / (_ROWS * _D)


# speedup vs baseline: 6.8577x; 1.0213x over previous
import functools

import jax
import jax.numpy as jnp
from jax import lax
from jax.experimental import pallas as pl
from jax.experimental.pallas import tpu as pltpu
from jax.experimental.pallas import tpu_sc as plsc

_B, _S, _DIM = 4, 2048, 6
_VOCAB, _D = 512, 1024
_DP = _D // 2
_ROWS = _B * (_S - 1)
_BLK = 512
_NBLK = 16
_NPOS = _NBLK * _BLK
_RPB = _DIM * _BLK

_KOH = 10
_NSC = _NBLK - _KOH
_NIDX = _NSC * _RPB

_NC, _NS = 2, 16
_NW = _NC * _NS
_BPW = _NIDX // _NW
_CH = 96
_NCH = _BPW // _CH

assert _BPW * _NW == _NIDX and _NCH * _CH == _BPW


def _sc_gather(table_hbm, idx_hbm, out_hbm, idx_v, rows_a, rows_b,
               sem_a, sem_b, wsem_a, wsem_b):
    wid = lax.axis_index("s") * _NC + lax.axis_index("c")
    base = wid * _BPW
    pltpu.sync_copy(idx_hbm.at[pl.ds(base, _BPW)], idx_v)
    bufs = (rows_a, rows_b)
    sems = (sem_a, sem_b)
    wsems = (wsem_a, wsem_b)

    def _start(c):
        pltpu.async_copy(
            table_hbm.at[idx_v.at[pl.ds(c * _CH, _CH)]],
            bufs[c % 2], sems[c % 2])

    def _wait_gather(c):
        pltpu.make_async_copy(
            table_hbm.at[idx_v.at[pl.ds(c * _CH, _CH)]],
            bufs[c % 2], sems[c % 2]).wait()

    def _start_write(c):
        pltpu.async_copy(
            bufs[c % 2], out_hbm.at[pl.ds(base + c * _CH, _CH)],
            wsems[c % 2])

    def _wait_write(c):
        pltpu.make_async_copy(
            bufs[c % 2], out_hbm.at[pl.ds(base + c * _CH, _CH)],
            wsems[c % 2]).wait()

    _start(0)
    if _NCH > 1:
        _start(1)
    for c in range(_NCH):
        _wait_gather(c)
        _start_write(c)
        if c + 2 < _NCH:
            _wait_write(c)
            _start(c + 2)
    if _NCH > 1:
        _wait_write(_NCH - 2)
    _wait_write(_NCH - 1)


def _prep_kernel(emb_ref, pack_ref, f8_ref):
    e = emb_ref[0]
    bits = lax.bitcast_convert_type(e, jnp.int32)
    pack_ref[...] = (lax.shift_right_logical(bits[:, :_DP], 16)
                     | (bits[:, _DP:] & jnp.int32(-65536)))
    f8_ref[0] = (e * 8.0).astype(jnp.float8_e4m3fn)


def _onehot_kernel(idx_ref, emb_ref, w_ref, out_ref):
    i = pl.program_id(0)

    h = jnp.zeros((_BLK, _D), dtype=jnp.float32)
    for d in range(_DIM):
        ids = idx_ref[0, d].reshape(_BLK, 1)
        oh = (jax.lax.broadcasted_iota(jnp.int32, (_BLK, _VOCAB), 1)
              == ids).astype(jnp.float8_e4m3fn)
        h = h + jnp.dot(oh, emb_ref[d], preferred_element_type=jnp.float32)

    t = (jnp.tanh(h * 0.125) * 8.0).astype(jnp.float8_e4m3fn)
    o = jnp.dot(t, w_ref[...], preferred_element_type=jnp.float32) * (1.0 / 128.0)
    diff = o - 1.0
    s = jnp.sum(diff * diff, keepdims=True)

    @pl.when(i == 0)
    def _():
        out_ref[...] = jnp.zeros((1, 1), jnp.float32)

    out_ref[...] += s


def _consume_kernel(g_ref, w_ref, out_ref):
    i = pl.program_id(0)

    he = jnp.zeros((_BLK, _DP), dtype=jnp.float32)
    ho = jnp.zeros((_BLK, _DP), dtype=jnp.float32)
    for d in range(_DIM):
        gd = g_ref[d * _BLK:(d + 1) * _BLK, :]
        he = he + lax.bitcast_convert_type(gd << 16, jnp.float32)
        ho = ho + lax.bitcast_convert_type(gd & jnp.int32(-65536),
                                           jnp.float32)

    t = (jnp.concatenate([jnp.tanh(he), jnp.tanh(ho)], axis=1)
         * 8.0).astype(jnp.float8_e4m3fn)
    o = jnp.dot(t, w_ref[...],
                preferred_element_type=jnp.float32) * (1.0 / 128.0)
    diff = o - 1.0

    row = (_KOH + i) * _BLK + jax.lax.broadcasted_iota(
        jnp.int32, (_BLK, _D), 0)
    diff = jnp.where(row < _ROWS, diff, 0.0)
    s = jnp.sum(diff * diff, keepdims=True)

    @pl.when(i == 0)
    def _():
        out_ref[...] = jnp.zeros((1, 1), jnp.float32)

    out_ref[...] += s


def kernel(x, emb, W):
    xi = x[:, :-1].reshape(_ROWS, _DIM).astype(jnp.int32)
    idx = jnp.pad(xi, ((0, _NPOS - _ROWS), (0, 0)))
    idx3 = (idx.reshape(_NBLK, _BLK, _DIM)
            .transpose(0, 2, 1))
    offs = (jnp.arange(_DIM, dtype=jnp.int32) * _VOCAB)[None, :, None]
    idx_sc = (idx3[_KOH:] + offs).reshape(_NIDX)

    table, emb_f8 = pl.pallas_call(
        _prep_kernel,
        grid=(_DIM,),
        in_specs=[pl.BlockSpec((1, _VOCAB, _D), lambda i: (i, 0, 0))],
        out_specs=[
            pl.BlockSpec((_VOCAB, _DP), lambda i: (i, 0)),
            pl.BlockSpec((1, _VOCAB, _D), lambda i: (i, 0, 0)),
        ],
        out_shape=[
            jax.ShapeDtypeStruct((_DIM * _VOCAB, _DP), jnp.int32),
            jax.ShapeDtypeStruct((_DIM, _VOCAB, _D), jnp.float8_e4m3fn),
        ],
    )(emb)

    w_f8 = (W * 16.0).astype(jnp.float8_e4m3fn)

    mesh = plsc.VectorSubcoreMesh(core_axis_name="c", subcore_axis_name="s")
    gather = functools.partial(
        pl.kernel,
        mesh=mesh,
        out_type=jax.ShapeDtypeStruct((_NIDX, _DP), jnp.int32),
        scratch_types=[
            pltpu.VMEM((_BPW,), jnp.int32),
            pltpu.VMEM((_CH, _DP), jnp.int32),
            pltpu.VMEM((_CH, _DP), jnp.int32),
            pltpu.SemaphoreType.DMA,
            pltpu.SemaphoreType.DMA,
            pltpu.SemaphoreType.DMA,
            pltpu.SemaphoreType.DMA,
        ],
    )(_sc_gather)
    g = gather(table, idx_sc)

    s_oh = pl.pallas_call(
        _onehot_kernel,
        grid=(_KOH,),
        in_specs=[
            pl.BlockSpec((1, _DIM, _BLK), lambda i: (i, 0, 0)),
            pl.BlockSpec((_DIM, _VOCAB, _D), lambda i: (0, 0, 0)),
            pl.BlockSpec((_D, _D), lambda i: (0, 0)),
        ],
        out_specs=pl.BlockSpec((1, 1), lambda i: (0, 0)),
        out_shape=jax.ShapeDtypeStruct((1, 1), jnp.float32),
    )(idx3[:_KOH], emb_f8, w_f8)

    s_sc = pl.pallas_call(
        _consume_kernel,
        grid=(_NSC,),
        in_specs=[
            pl.BlockSpec((_RPB, _DP), lambda i: (i, 0)),
            pl.BlockSpec((_D, _D), lambda i: (0, 0)),
        ],
        out_specs=pl.BlockSpec((1, 1), lambda i: (0, 0)),
        out_shape=jax.ShapeDtypeStruct((1, 1), jnp.float32),
    )(g, w_f8)

    return (s_oh[0, 0] + s_sc[0, 0]) / (_ROWS * _D)
